# baseline (speedup 1.0000x reference)
import jax
import jax.numpy as jnp
from jax import lax
from jax.experimental import pallas as pl
from jax.experimental.pallas import tpu as pltpu
from jax.experimental.pallas import tpu_sc as plsc

_ROWS, _DIM = 41, 77


def _sc_body(exp_hbm, e_hbm, out_hbm, exp_s):
    pltpu.sync_copy(exp_hbm, exp_s)
    e = exp_s[0]
    idx = jnp.clip(e, -20, 20) + 20
    pltpu.sync_copy(e_hbm.at[pl.ds(idx, 1)], out_hbm)


def kernel(exponent, E):
    exp_arr = jnp.asarray(exponent, jnp.int32).reshape((1,))
    run = pl.kernel(
        _sc_body,
        out_type=jax.ShapeDtypeStruct((1, _DIM), jnp.float32),
        mesh=plsc.ScalarSubcoreMesh(axis_name="c", num_cores=1),
        scratch_types=[
            pltpu.SMEM((1,), jnp.int32),
        ],
    )
    return run(exp_arr, E).reshape((_DIM,))


# speedup vs baseline: 1.0065x; 1.0065x over previous
import jax
import jax.numpy as jnp
from jax.experimental import pallas as pl
from jax.experimental.pallas import tpu as pltpu
from jax.experimental.pallas import tpu_sc as plsc

_ROWS, _DIM = 41, 77


def _sc_body(exp_hbm, e_hbm, out_hbm, exp_s):
    pltpu.sync_copy(exp_hbm, exp_s)
    e = exp_s[0]
    idx = jnp.clip(e, -20, 20) + 20
    pltpu.sync_copy(e_hbm.at[pl.ds(idx, 1)], out_hbm)


def kernel(exponent, E):
    exp_arr = jnp.asarray(exponent, jnp.int32).reshape((1,))
    run = pl.kernel(
        _sc_body,
        out_type=jax.ShapeDtypeStruct((1, _DIM), jnp.float32),
        mesh=plsc.ScalarSubcoreMesh(axis_name="c", num_cores=1),
        scratch_types=[
            pltpu.SMEM((1,), jnp.int32),
        ],
    )
    return run(exp_arr, E).reshape((_DIM,))
